# baseline (speedup 1.0000x reference)
import functools

import jax
import jax.numpy as jnp
from jax import lax
from jax.experimental import pallas as pl
from jax.experimental.pallas import tpu as pltpu
from jax.experimental.pallas import tpu_sc as plsc

_N_FIELDS = 26
_VOCAB = 100000
_EMB = 16
_NUM_CORES = 2
_NUM_SUBCORES = 16
_LANES = 16


@functools.partial(jax.jit, static_argnums=(2, 3))
def _embedding_gather(x_flat, tab_flat, batch, bchunk):
    n_workers = _NUM_CORES * _NUM_SUBCORES
    b_per_w = batch // n_workers
    rows_per_w = b_per_w * _N_FIELDS
    chunk = bchunk * _N_FIELDS
    n_chunks = b_per_w // bchunk
    qgroups = bchunk // _LANES
    mesh = plsc.VectorSubcoreMesh(core_axis_name="c", subcore_axis_name="s")

    def body(x_hbm, tab_hbm, out_hbm, xv, idx_t, rows_v, gsem, wsem):
        wid = lax.axis_index("s") * _NUM_CORES + lax.axis_index("c")
        wbase = wid * rows_per_w
        lanes = lax.iota(jnp.int32, _LANES)

        pltpu.sync_copy(x_hbm.at[pl.ds(wbase, rows_per_w)], xv)

        def do_chunk(g, carry):
            cbase = g * chunk
            for f in range(_N_FIELDS):
                for q in range(qgroups):
                    src = (cbase + (q * _LANES + lanes) * _N_FIELDS) + f
                    vals = plsc.load_gather(xv, [src])
                    idx_t[pl.ds(f * bchunk + q * _LANES, _LANES)] = (
                        vals + f * _VOCAB)

            pltpu.async_copy(tab_hbm.at[idx_t], rows_v, gsem).wait()

            b0 = wid * b_per_w + g * bchunk
            copies = []
            for f in range(_N_FIELDS):
                copies.append(pltpu.async_copy(
                    rows_v.at[pl.ds(f * bchunk, bchunk), :],
                    out_hbm.at[pl.ds(b0, bchunk), pl.ds(f * _EMB, _EMB)],
                    wsem))
            for c in copies:
                c.wait()
            return carry

        lax.fori_loop(0, n_chunks, do_chunk, 0)

    return pl.kernel(
        body,
        out_type=jax.ShapeDtypeStruct((batch, _N_FIELDS * _EMB), jnp.float32),
        mesh=mesh,
        scratch_types=[
            pltpu.VMEM((rows_per_w,), jnp.int32),
            pltpu.VMEM((chunk,), jnp.int32),
            pltpu.VMEM((chunk, _EMB), jnp.float32),
            pltpu.SemaphoreType.DMA,
            pltpu.SemaphoreType.DMA,
        ],
        compiler_params=pltpu.CompilerParams(use_tc_tiling_on_sc=False,
                                             needs_layout_passes=False),
    )(x_flat, tab_flat)


def kernel(x, tables):
    batch, n_fields = x.shape
    x_flat = x.reshape(batch * n_fields)
    tab_flat = tables.reshape(n_fields * _VOCAB, _EMB)
    return _embedding_gather(x_flat, tab_flat, batch, 64)
